# Initial kernel scaffold; baseline (speedup 1.0000x reference)
import functools

import jax
import jax.numpy as jnp
from jax import lax
from jax.experimental import pallas as pl
from jax.experimental.pallas import tpu as pltpu
from jax.experimental.pallas import tpu_sc as plsc

_B, _S = 16384, 50
_D = 32
_TOTAL = _B * _S
_NW = 32
_CHUNK = 128
_NCHUNKS = _TOTAL // _CHUNK
_PER_W = _NCHUNKS // _NW


def _body(tok_hbm, table_hbm, out_hbm, idx_v, rows_v, sem):
    cid = lax.axis_index("c")
    sid = lax.axis_index("s")
    wid = sid * 2 + cid
    base = wid * _PER_W
    pltpu.sync_copy(tok_hbm.at[pl.ds(base, _PER_W)], idx_v)

    def step(j, carry):
        pltpu.async_copy(table_hbm.at[idx_v.at[j]], rows_v, sem).wait()
        pltpu.sync_copy(rows_v, out_hbm.at[base + j])
        return carry

    lax.fori_loop(0, _PER_W, step, 0)


@jax.jit
def _embed(tok2d, weight):
    mesh = plsc.VectorSubcoreMesh(core_axis_name="c", subcore_axis_name="s")
    kern = functools.partial(
        pl.kernel,
        mesh=mesh,
        out_type=jax.ShapeDtypeStruct((_NCHUNKS, _CHUNK, _D), jnp.float32),
        scratch_types=[
            pltpu.VMEM((_PER_W, _CHUNK), jnp.int32),
            pltpu.VMEM((_CHUNK, _D), jnp.float32),
            pltpu.SemaphoreType.DMA,
        ],
    )(_body)
    return kern(tok2d, weight)


def kernel(token_ids, weight):
    tok = token_ids.reshape(_NCHUNKS, _CHUNK).astype(jnp.int32)
    out = _embed(tok, weight)
    return out.reshape(_B, _S, _D)


# speedup vs baseline: 1.1878x; 1.1878x over previous
import functools

import jax
import jax.numpy as jnp
from jax import lax
from jax.experimental import pallas as pl
from jax.experimental.pallas import tpu as pltpu
from jax.experimental.pallas import tpu_sc as plsc

_B, _S = 16384, 50
_D = 32
_TOTAL = _B * _S
_NW = 32
_CHUNK = 128
_NCHUNKS = _TOTAL // _CHUNK
_PER_W = _NCHUNKS // _NW


def _body(tok_hbm, table_hbm, out_hbm, idx_v, rows_v, sem):
    cid = lax.axis_index("c")
    sid = lax.axis_index("s")
    wid = sid * 2 + cid
    base = wid * _PER_W
    pltpu.sync_copy(tok_hbm.at[pl.ds(base, _PER_W)], idx_v)

    def step(j, carry):
        pltpu.async_copy(table_hbm.at[idx_v.at[j]], rows_v, sem).wait()
        pltpu.sync_copy(rows_v, out_hbm.at[base + j])
        return carry

    lax.fori_loop(0, _PER_W, step, 0)


@jax.jit
def _embed(tok2d, weight):
    mesh = plsc.VectorSubcoreMesh(core_axis_name="c", subcore_axis_name="s")
    kern = functools.partial(
        pl.kernel,
        mesh=mesh,
        out_type=jax.ShapeDtypeStruct((_NCHUNKS, _CHUNK, _D), jnp.float32),
        scratch_types=[
            pltpu.VMEM((_PER_W, _CHUNK), jnp.int32),
            pltpu.VMEM((_CHUNK, _D), jnp.float32),
            pltpu.SemaphoreType.DMA,
        ],
        compiler_params=pltpu.CompilerParams(use_tc_tiling_on_sc=False),
    )(_body)
    return kern(tok2d, weight)


def kernel(token_ids, weight):
    tok = token_ids.reshape(_NCHUNKS, _CHUNK).astype(jnp.int32)
    out = _embed(tok, weight)
    return out.reshape(_B, _S, _D)
